# baseline (speedup 1.0000x reference)
import functools

import jax
import jax.numpy as jnp
from jax import lax
from jax.experimental import pallas as pl
from jax.experimental.pallas import tpu as pltpu
from jax.experimental.pallas import tpu_sc as plsc

VOCAB_SIZE = 100
NUM_IDS = 327680
SPLIT = 48
SC_ROWS = VOCAB_SIZE - SPLIT

NUM_CORES = 2
NUM_SUBCORES = 16
NUM_WORKERS = NUM_CORES * NUM_SUBCORES
TOKENS_PER_WORKER = NUM_IDS // NUM_WORKERS
CHUNK_TOKENS = 1024
NUM_ROUNDS = TOKENS_PER_WORKER // CHUNK_TOKENS
CHUNK_WORDS = CHUNK_TOKENS * SC_ROWS
LANES = 16

TC_COLS = 8192
TC_BLOCKS = NUM_IDS // TC_COLS


def _sc_body(ids_hbm, out_hbm, idx0, idx1, cols0, cols1, sem0, sem1):
    wid = lax.axis_index("s") * NUM_CORES + lax.axis_index("c")
    tok_base = wid * TOKENS_PER_WORKER
    idx = [idx0, idx1]
    cols = [cols0, cols1]
    sem = [sem0, sem1]

    lane = lax.iota(jnp.int32, LANES)
    ones = jnp.full((LANES,), 1, jnp.int32)
    zeros = jnp.full((LANES,), 0, jnp.int32)

    def _zinit(buf):
        def body(i, _):
            flat = i * LANES + lane
            plsc.store_scatter(
                buf, [flat // CHUNK_TOKENS, flat % CHUNK_TOKENS], zeros
            )
            return 0

        lax.fori_loop(0, CHUNK_WORDS // LANES, body, 0)

    def _scatter(buf, ids_ref, val):
        def body(j, _):
            ids16 = ids_ref[pl.ds(j * LANES, LANES)]
            plsc.store_scatter(
                buf, [ids16 - SPLIT, j * LANES + lane], val, mask=ids16 >= SPLIT
            )
            return 0

        lax.fori_loop(0, CHUNK_TOKENS // LANES, body, 0)

    def _out_slice(r):
        return out_hbm.at[
            pl.ds(SPLIT, SC_ROWS),
            pl.ds(tok_base + r * CHUNK_TOKENS, CHUNK_TOKENS),
        ]

    _zinit(cols0)
    _zinit(cols1)
    pltpu.sync_copy(ids_hbm.at[pl.ds(tok_base, CHUNK_TOKENS)], idx0)

    pending = [None, None]
    for r in range(NUM_ROUNDS):
        b = r % 2
        nb = 1 - b
        _scatter(cols[b], idx[b], ones)
        pltpu.make_async_copy(cols[b], _out_slice(r), sem[b]).start()
        pending[b] = r
        if pending[nb] is not None:
            pltpu.make_async_copy(cols[nb], _out_slice(pending[nb]), sem[nb]).wait()
            if r + 1 < NUM_ROUNDS:
                _scatter(cols[nb], idx[nb], zeros)
            pending[nb] = None
        if r + 1 < NUM_ROUNDS:
            tok0 = tok_base + (r + 1) * CHUNK_TOKENS
            pltpu.sync_copy(ids_hbm.at[pl.ds(tok0, CHUNK_TOKENS)], idx[nb])
    for b in range(2):
        if pending[b] is not None:
            pltpu.make_async_copy(cols[b], _out_slice(pending[b]), sem[b]).wait()


_sc_call = functools.partial(
    pl.kernel,
    out_type=jax.ShapeDtypeStruct((VOCAB_SIZE, NUM_IDS), jnp.int32),
    mesh=plsc.VectorSubcoreMesh(core_axis_name="c", subcore_axis_name="s"),
    scratch_types=[
        pltpu.VMEM((CHUNK_TOKENS,), jnp.int32),
        pltpu.VMEM((CHUNK_TOKENS,), jnp.int32),
        pltpu.VMEM((SC_ROWS, CHUNK_TOKENS), jnp.int32),
        pltpu.VMEM((SC_ROWS, CHUNK_TOKENS), jnp.int32),
        pltpu.SemaphoreType.DMA,
        pltpu.SemaphoreType.DMA,
    ],
    compiler_params=pltpu.CompilerParams(needs_layout_passes=False),
)(_sc_body)


def _tc_block(ids_ref, t_ref, out_ref):
    del t_ref
    rows = lax.broadcasted_iota(jnp.int32, (SPLIT, TC_COLS), 0)
    out_ref[...] = (rows == ids_ref[0]).astype(jnp.int32)


def _tc_fill(ids, t):
    ids3 = ids.reshape(TC_BLOCKS, 1, TC_COLS)
    return pl.pallas_call(
        _tc_block,
        grid=(TC_BLOCKS,),
        in_specs=[
            pl.BlockSpec((1, 1, TC_COLS), lambda i: (i, 0, 0)),
            pl.BlockSpec(memory_space=pl.ANY),
        ],
        out_specs=pl.BlockSpec((SPLIT, TC_COLS), lambda i: (0, i)),
        out_shape=jax.ShapeDtypeStruct((VOCAB_SIZE, NUM_IDS), jnp.int32),
        input_output_aliases={1: 0},
        compiler_params=pltpu.CompilerParams(
            dimension_semantics=("arbitrary",),
        ),
    )(ids3, t)


def kernel(input):
    t = _sc_call(input)
    return _tc_fill(input, t).T


# speedup vs baseline: 1.2445x; 1.2445x over previous
import functools

import jax
import jax.numpy as jnp
from jax import lax
from jax.experimental import pallas as pl
from jax.experimental.pallas import tpu as pltpu
from jax.experimental.pallas import tpu_sc as plsc

VOCAB_SIZE = 100
NUM_IDS = 327680
NUM_CORES = 2
NUM_SUBCORES = 16
NUM_WORKERS = NUM_CORES * NUM_SUBCORES
TOKENS_PER_WORKER = NUM_IDS // NUM_WORKERS
CHUNK_TOKENS = 512
NUM_ROUNDS = TOKENS_PER_WORKER // CHUNK_TOKENS
CHUNK_WORDS = CHUNK_TOKENS * VOCAB_SIZE
LANES = 16
UNROLL = 4


def _sc_body(ids_hbm, out_hbm, idx0, idx1, idx2, cols0, cols1, sem0, sem1):
    wid = lax.axis_index("s") * NUM_CORES + lax.axis_index("c")
    tok_base = wid * TOKENS_PER_WORKER
    idx = [idx0, idx1, idx2]
    cols = [cols0, cols1]
    sem = [sem0, sem1]

    lane = lax.iota(jnp.int32, LANES)
    ones = jnp.full((LANES,), 1, jnp.int32)
    zeros = jnp.full((LANES,), 0, jnp.int32)

    def _zinit(buf):
        def body(i, _):
            for u in range(UNROLL):
                flat = (i * UNROLL + u) * LANES + lane
                plsc.store_scatter(
                    buf, [flat // CHUNK_TOKENS, flat % CHUNK_TOKENS], zeros
                )
            return 0

        lax.fori_loop(0, CHUNK_WORDS // (LANES * UNROLL), body, 0)

    def _scatter(buf, ids_ref, val):
        def body(j, _):
            for u in range(UNROLL):
                t = (j * UNROLL + u) * LANES
                ids16 = ids_ref[pl.ds(t, LANES)]
                plsc.store_scatter(buf, [ids16, t + lane], val)
            return 0

        lax.fori_loop(0, CHUNK_TOKENS // (LANES * UNROLL), body, 0)

    def _load_ids(r):
        tok0 = tok_base + r * CHUNK_TOKENS
        pltpu.sync_copy(ids_hbm.at[pl.ds(tok0, CHUNK_TOKENS)], idx[r % 3])

    def _out_slice(r):
        return out_hbm.at[:, pl.ds(tok_base + r * CHUNK_TOKENS, CHUNK_TOKENS)]

    _zinit(cols0)
    _load_ids(0)

    pending = [None, None]
    for r in range(NUM_ROUNDS):
        b = r % 2
        nb = 1 - b
        _scatter(cols[b], idx[r % 3], ones)
        pltpu.make_async_copy(cols[b], _out_slice(r), sem[b]).start()
        pending[b] = r
        if r == 0:
            _zinit(cols1)
        if r + 1 < NUM_ROUNDS:
            _load_ids(r + 1)
        if pending[nb] is not None:
            pltpu.make_async_copy(cols[nb], _out_slice(pending[nb]), sem[nb]).wait()
            if r + 1 < NUM_ROUNDS:
                _scatter(cols[nb], idx[pending[nb] % 3], zeros)
            pending[nb] = None
    for b in range(2):
        if pending[b] is not None:
            pltpu.make_async_copy(cols[b], _out_slice(pending[b]), sem[b]).wait()


_sc_call = functools.partial(
    pl.kernel,
    out_type=jax.ShapeDtypeStruct((VOCAB_SIZE, NUM_IDS), jnp.int32),
    mesh=plsc.VectorSubcoreMesh(core_axis_name="c", subcore_axis_name="s"),
    scratch_types=[
        pltpu.VMEM((CHUNK_TOKENS,), jnp.int32),
        pltpu.VMEM((CHUNK_TOKENS,), jnp.int32),
        pltpu.VMEM((CHUNK_TOKENS,), jnp.int32),
        pltpu.VMEM((VOCAB_SIZE, CHUNK_TOKENS), jnp.int32),
        pltpu.VMEM((VOCAB_SIZE, CHUNK_TOKENS), jnp.int32),
        pltpu.SemaphoreType.DMA,
        pltpu.SemaphoreType.DMA,
    ],
    compiler_params=pltpu.CompilerParams(needs_layout_passes=False),
)(_sc_body)


def kernel(input):
    return _sc_call(input).T
